# baseline (speedup 1.0000x reference)
import jax
import jax.numpy as jnp
from jax import lax
from jax.experimental import pallas as pl
from jax.experimental.pallas import tpu as pltpu
from jax.experimental.pallas import tpu_sc as plsc

NUM_MODELS = 2
NUM_BOXES = 1000000
DIM = 32
BATCH = 4096

_INFO = plsc.get_sparse_core_info()
_NC = _INFO.num_cores
_NS = _INFO.num_subcores
_NW = _NC * _NS
_L = _INFO.num_lanes

_VROWS = NUM_MODELS * BATCH
_RPW = _VROWS // _NW


def _sc_body(z_hbm, ld_hbm, idx_hbm, out_hbm, idx_v, zbuf, lbuf, obuf,
             sem_z, sem_l):
    wid = lax.axis_index("c") * _NS + lax.axis_index("s")
    model = wid // (BATCH // _RPW)
    b0 = (wid % (BATCH // _RPW)) * _RPW

    pltpu.sync_copy(idx_hbm.at[pl.ds(b0, _RPW)], idx_v)
    off = model * NUM_BOXES
    for i in range(_RPW // _L):
        sl = pl.ds(i * _L, _L)
        idx_v[sl] = idx_v[sl] + off

    cz = pltpu.async_copy(z_hbm.at[idx_v], zbuf, sem_z)
    cl = pltpu.async_copy(ld_hbm.at[idx_v], lbuf, sem_l)
    cz.wait()
    cl.wait()

    def row(r, carry):
        for j in range(DIM // _L):
            sl = pl.ds(j * _L, _L)
            zv = zbuf[r, sl]
            ev = zv + jnp.exp(lbuf[r, sl])
            obuf[r, pl.ds(j * _L, _L)] = zv
            obuf[r, pl.ds(DIM + j * _L, _L)] = ev
        return carry

    lax.fori_loop(0, _RPW, row, 0)

    pltpu.sync_copy(obuf, out_hbm.at[pl.ds(wid * _RPW, _RPW)])


@jax.jit
def kernel(z, logdelta, box_indices):
    zf = z.reshape(NUM_MODELS * NUM_BOXES, DIM)
    lf = logdelta.reshape(NUM_MODELS * NUM_BOXES, DIM)
    idx = box_indices.astype(jnp.int32)

    mesh = plsc.VectorSubcoreMesh(core_axis_name="c", subcore_axis_name="s")
    out = pl.kernel(
        _sc_body,
        mesh=mesh,
        compiler_params=pltpu.CompilerParams(use_tc_tiling_on_sc=False),
        out_type=jax.ShapeDtypeStruct((_VROWS, 2 * DIM), jnp.float32),
        scratch_types=[
            pltpu.VMEM((_RPW,), jnp.int32),
            pltpu.VMEM((_RPW, DIM), jnp.float32),
            pltpu.VMEM((_RPW, DIM), jnp.float32),
            pltpu.VMEM((_RPW, 2 * DIM), jnp.float32),
            pltpu.SemaphoreType.DMA,
            pltpu.SemaphoreType.DMA,
        ],
    )(zf, lf, idx)
    return out.reshape(NUM_MODELS, BATCH, 2, DIM)


# speedup vs baseline: 13.5522x; 13.5522x over previous
import jax
import jax.numpy as jnp
from jax import lax
from jax.experimental import pallas as pl
from jax.experimental.pallas import tpu as pltpu
from jax.experimental.pallas import tpu_sc as plsc

NUM_MODELS = 2
NUM_BOXES = 1000000
DIM = 32
BATCH = 4096

_INFO = plsc.get_sparse_core_info()
_NC = _INFO.num_cores
_NS = _INFO.num_subcores
_NW = _NC * _NS
_L = _INFO.num_lanes

_P = NUM_MODELS * DIM
_BPW = BATCH // _NW
_PAIRS = _BPW // 2

_GROUP_ROW0 = (0, 16, 64, 80)


def _sc_body(z_hbm, ld_hbm, idx_hbm, out_hbm, idx_v, zblk, lblk, obuf,
             semz0, semz1, seml0, seml1):
    wid = lax.axis_index("c") * _NS + lax.axis_index("s")
    b0 = wid * _BPW

    pltpu.sync_copy(idx_hbm.at[pl.ds(b0, _BPW)], idx_v.at[pl.ds(0, _BPW)])

    iota = lax.iota(jnp.int32, _L)
    semz = (semz0, semz1)
    seml = (seml0, seml1)

    def enqueue(pair, slot):
        chunk = idx_v[pl.ds(2 * pair, _L)]
        for k in range(2):
            c0 = pl.multiple_of((chunk[k] >> 7) << 7, 128)
            pltpu.async_copy(z_hbm.at[:, pl.ds(c0, 128)],
                             zblk.at[2 * slot + k], semz[slot])
            pltpu.async_copy(ld_hbm.at[:, pl.ds(c0, 128)],
                             lblk.at[2 * slot + k], seml[slot])

    def drain(slot):
        for k in range(2):
            pltpu.make_async_copy(z_hbm.at[:, pl.ds(0, 128)],
                                  zblk.at[2 * slot + k], semz[slot]).wait()
            pltpu.make_async_copy(ld_hbm.at[:, pl.ds(0, 128)],
                                  lblk.at[2 * slot + k], seml[slot]).wait()

    enqueue(0, 0)

    def step(i2, carry):
        for slot in range(2):
            pair = 2 * i2 + slot

            @pl.when(pair < _PAIRS - 1)
            def _(pair=pair, slot=slot):
                enqueue(pair + 1, 1 - slot)

            drain(slot)

            chunk = idx_v[pl.ds(2 * pair, _L)]
            for k in range(2):
                col = jnp.full((_L,), chunk[k] & 127, jnp.int32)
                kk = jnp.full((_L,), 2 * slot + k, jnp.int32)
                jj = jnp.full((_L,), 2 * pair + k, jnp.int32)
                for g in range(_P // _L):
                    rows = iota + g * _L
                    zv = plsc.load_gather(zblk, [kk, rows, col])
                    lv = plsc.load_gather(lblk, [kk, rows, col])
                    ev = zv + jnp.exp(lv)
                    orow = iota + _GROUP_ROW0[g]
                    plsc.store_scatter(obuf, [orow, jj], zv)
                    plsc.store_scatter(obuf, [orow + DIM, jj], ev)
        return carry

    lax.fori_loop(0, _PAIRS // 2, step, 0)

    pltpu.sync_copy(obuf, out_hbm.at[:, pl.ds(b0, _BPW)])


@jax.jit
def kernel(z, logdelta, box_indices):
    zf = z.transpose(0, 2, 1).reshape(_P, NUM_BOXES)
    lf = logdelta.transpose(0, 2, 1).reshape(_P, NUM_BOXES)
    idx = box_indices.astype(jnp.int32)

    mesh = plsc.VectorSubcoreMesh(core_axis_name="c", subcore_axis_name="s")
    out = pl.kernel(
        _sc_body,
        mesh=mesh,
        compiler_params=pltpu.CompilerParams(needs_layout_passes=False),
        out_type=jax.ShapeDtypeStruct((2 * _P, BATCH), jnp.float32),
        scratch_types=[
            pltpu.VMEM((_BPW + _L,), jnp.int32),
            pltpu.VMEM((4, _P, 128), jnp.float32),
            pltpu.VMEM((4, _P, 128), jnp.float32),
            pltpu.VMEM((2 * _P, _BPW), jnp.float32),
            pltpu.SemaphoreType.DMA,
            pltpu.SemaphoreType.DMA,
            pltpu.SemaphoreType.DMA,
            pltpu.SemaphoreType.DMA,
        ],
    )(zf, lf, idx)
    return out.reshape(NUM_MODELS, 2, DIM, BATCH).transpose(0, 3, 1, 2)


# speedup vs baseline: 14.8034x; 1.0923x over previous
import jax
import jax.numpy as jnp
from jax import lax
from jax.experimental import pallas as pl
from jax.experimental.pallas import tpu as pltpu
from jax.experimental.pallas import tpu_sc as plsc

NUM_MODELS = 2
NUM_BOXES = 1000000
DIM = 32
BATCH = 4096

_INFO = plsc.get_sparse_core_info()
_NC = _INFO.num_cores
_NS = _INFO.num_subcores
_NW = _NC * _NS
_L = _INFO.num_lanes

_P = NUM_MODELS * DIM
_BPW = BATCH // _NW
_PAIRS = _BPW // 2

_GROUP_ROW0 = (0, 16, 64, 80)


def _sc_body(z_hbm, ld_hbm, idx_hbm, out_hbm, idx_v, zblk, lblk, obuf,
             semz0, semz1, semz2, semz3, seml0, seml1, seml2, seml3):
    wid = lax.axis_index("c") * _NS + lax.axis_index("s")
    b0 = wid * _BPW

    pltpu.sync_copy(idx_hbm.at[pl.ds(b0, _BPW)], idx_v.at[pl.ds(0, _BPW)])

    iota = lax.iota(jnp.int32, _L)
    semz = (semz0, semz1, semz2, semz3)
    seml = (seml0, seml1, seml2, seml3)
    _NSLOT = 4

    def enqueue(box, slot):
        chunk = idx_v[pl.ds(box, _L)]
        c0 = pl.multiple_of((chunk[0] >> 7) << 7, 128)
        pltpu.async_copy(z_hbm.at[:, pl.ds(c0, 128)],
                         zblk.at[slot], semz[slot])
        pltpu.async_copy(ld_hbm.at[:, pl.ds(c0, 128)],
                         lblk.at[slot], seml[slot])

    def drain(slot):
        pltpu.make_async_copy(z_hbm.at[:, pl.ds(0, 128)],
                              zblk.at[slot], semz[slot]).wait()
        pltpu.make_async_copy(ld_hbm.at[:, pl.ds(0, 128)],
                              lblk.at[slot], seml[slot]).wait()

    for s in range(_NSLOT - 1):
        enqueue(s, s)

    def step(i4, carry):
        for slot in range(_NSLOT):
            box = _NSLOT * i4 + slot

            @pl.when(box < _BPW - (_NSLOT - 1))
            def _(box=box, slot=slot):
                enqueue(box + _NSLOT - 1, (slot + _NSLOT - 1) % _NSLOT)

            drain(slot)

            chunk = idx_v[pl.ds(box, _L)]
            col = jnp.full((_L,), chunk[0] & 127, jnp.int32)
            kk = jnp.full((_L,), slot, jnp.int32)
            jj = jnp.full((_L,), box, jnp.int32)
            for g in range(_P // _L):
                rows = iota + g * _L
                zv = plsc.load_gather(zblk, [kk, rows, col])
                lv = plsc.load_gather(lblk, [kk, rows, col])
                ev = zv + jnp.exp(lv)
                orow = iota + _GROUP_ROW0[g]
                plsc.store_scatter(obuf, [orow, jj], zv)
                plsc.store_scatter(obuf, [orow + DIM, jj], ev)
        return carry

    lax.fori_loop(0, _BPW // _NSLOT, step, 0)

    pltpu.sync_copy(obuf, out_hbm.at[:, pl.ds(b0, _BPW)])


@jax.jit
def kernel(z, logdelta, box_indices):
    zf = z.transpose(0, 2, 1).reshape(_P, NUM_BOXES)
    lf = logdelta.transpose(0, 2, 1).reshape(_P, NUM_BOXES)
    idx = box_indices.astype(jnp.int32)

    mesh = plsc.VectorSubcoreMesh(core_axis_name="c", subcore_axis_name="s")
    out = pl.kernel(
        _sc_body,
        mesh=mesh,
        compiler_params=pltpu.CompilerParams(needs_layout_passes=False),
        out_type=jax.ShapeDtypeStruct((2 * _P, BATCH), jnp.float32),
        scratch_types=[
            pltpu.VMEM((_BPW + _L,), jnp.int32),
            pltpu.VMEM((4, _P, 128), jnp.float32),
            pltpu.VMEM((4, _P, 128), jnp.float32),
            pltpu.VMEM((2 * _P, _BPW), jnp.float32),
            pltpu.SemaphoreType.DMA,
            pltpu.SemaphoreType.DMA,
            pltpu.SemaphoreType.DMA,
            pltpu.SemaphoreType.DMA,
            pltpu.SemaphoreType.DMA,
            pltpu.SemaphoreType.DMA,
            pltpu.SemaphoreType.DMA,
            pltpu.SemaphoreType.DMA,
        ],
    )(zf, lf, idx)
    return out.reshape(NUM_MODELS, 2, DIM, BATCH).transpose(0, 3, 1, 2)


# speedup vs baseline: 14.8782x; 1.0051x over previous
import jax
import jax.numpy as jnp
from jax import lax
from jax.experimental import pallas as pl
from jax.experimental.pallas import tpu as pltpu
from jax.experimental.pallas import tpu_sc as plsc

NUM_MODELS = 2
NUM_BOXES = 1000000
DIM = 32
BATCH = 4096

_INFO = plsc.get_sparse_core_info()
_NC = _INFO.num_cores
_NS = _INFO.num_subcores
_NW = _NC * _NS
_L = _INFO.num_lanes

_P = NUM_MODELS * DIM
_BPW = BATCH // _NW
_NSLOT = 6
_MAIN = (_BPW // _NSLOT) * _NSLOT

_GROUP_ROW0 = (0, 16, 64, 80)


def _sc_body(z_hbm, ld_hbm, idx_hbm, out_hbm, idx_v, zblk, lblk, obuf, *sems):
    semz = sems[:_NSLOT]
    seml = sems[_NSLOT:]
    wid = lax.axis_index("c") * _NS + lax.axis_index("s")
    b0 = wid * _BPW

    pltpu.sync_copy(idx_hbm.at[pl.ds(b0, _BPW)], idx_v.at[pl.ds(0, _BPW)])

    iota = lax.iota(jnp.int32, _L)

    def enqueue(box, slot):
        chunk = idx_v[pl.ds(box, _L)]
        c0 = pl.multiple_of((chunk[0] >> 7) << 7, 128)
        pltpu.async_copy(z_hbm.at[:, pl.ds(c0, 128)], zblk.at[slot], semz[slot])
        pltpu.async_copy(ld_hbm.at[:, pl.ds(c0, 128)], lblk.at[slot], seml[slot])

    def drain(slot):
        pltpu.make_async_copy(z_hbm.at[:, pl.ds(0, 128)],
                              zblk.at[slot], semz[slot]).wait()
        pltpu.make_async_copy(ld_hbm.at[:, pl.ds(0, 128)],
                              lblk.at[slot], seml[slot]).wait()

    def compute(box, slot):
        chunk = idx_v[pl.ds(box, _L)]
        col = jnp.full((_L,), chunk[0] & 127, jnp.int32)
        kk = jnp.full((_L,), slot, jnp.int32)
        jj = jnp.full((_L,), box, jnp.int32)
        for g in range(_P // _L):
            rows = iota + g * _L
            zv = plsc.load_gather(zblk, [kk, rows, col])
            lv = plsc.load_gather(lblk, [kk, rows, col])
            ev = zv + jnp.exp(lv)
            orow = iota + _GROUP_ROW0[g]
            plsc.store_scatter(obuf, [orow, jj], zv)
            plsc.store_scatter(obuf, [orow + DIM, jj], ev)

    for s in range(_NSLOT - 1):
        enqueue(s, s)

    def step(it, carry):
        for slot in range(_NSLOT):
            box = _NSLOT * it + slot

            @pl.when(box + _NSLOT - 1 < _BPW)
            def _(box=box, slot=slot):
                enqueue(box + _NSLOT - 1, (slot + _NSLOT - 1) % _NSLOT)

            drain(slot)
            compute(box, slot)
        return carry

    lax.fori_loop(0, _MAIN // _NSLOT, step, 0)

    for box in range(_MAIN, _BPW):
        drain(box % _NSLOT)
        compute(box, box % _NSLOT)

    pltpu.sync_copy(obuf, out_hbm.at[:, pl.ds(b0, _BPW)])


@jax.jit
def kernel(z, logdelta, box_indices):
    zf = z.transpose(0, 2, 1).reshape(_P, NUM_BOXES)
    lf = logdelta.transpose(0, 2, 1).reshape(_P, NUM_BOXES)
    idx = box_indices.astype(jnp.int32)

    mesh = plsc.VectorSubcoreMesh(core_axis_name="c", subcore_axis_name="s")
    out = pl.kernel(
        _sc_body,
        mesh=mesh,
        compiler_params=pltpu.CompilerParams(needs_layout_passes=False),
        out_type=jax.ShapeDtypeStruct((2 * _P, BATCH), jnp.float32),
        scratch_types=[
            pltpu.VMEM((_BPW + _L,), jnp.int32),
            pltpu.VMEM((_NSLOT, _P, 128), jnp.float32),
            pltpu.VMEM((_NSLOT, _P, 128), jnp.float32),
            pltpu.VMEM((2 * _P, _BPW), jnp.float32),
        ] + [pltpu.SemaphoreType.DMA] * (2 * _NSLOT),
    )(zf, lf, idx)
    return out.reshape(NUM_MODELS, 2, DIM, BATCH).transpose(0, 3, 1, 2)
